# baseline (speedup 1.0000x reference)
import functools

import jax
import jax.numpy as jnp
from jax import lax
from jax.experimental import pallas as pl
from jax.experimental.pallas import tpu as pltpu

ALPHA = 0.25
BLK = 25000

_CONTRACT0 = (((0,), (0,)), ((), ()))
_HI = lax.Precision.HIGHEST


def _focal_body(cls_ref, reg_ref, anc_ref, ann_ref,
                cls_out, reg_out, npos_out):
    b = pl.program_id(1)

    ann = ann_ref[0]
    x1 = ann[:, 0:1]
    y1 = ann[:, 1:2]
    x2 = ann[:, 2:3]
    y2 = ann[:, 3:4]
    bcls = ann[:, 4:5]
    area_b = (x2 - x1) * (y2 - y1)

    anc = anc_ref[0]
    ax1 = anc[0:1, :]
    ay1 = anc[1:2, :]
    ax2 = anc[2:3, :]
    ay2 = anc[3:4, :]
    aw = ax2 - ax1
    ah = ay2 - ay1
    acx = ax1 + 0.5 * aw
    acy = ay1 + 0.5 * ah

    iw = jnp.clip(jnp.minimum(ax2, x2) - jnp.maximum(ax1, x1), 0.0, None)
    ih = jnp.clip(jnp.minimum(ay2, y2) - jnp.maximum(ay1, y1), 0.0, None)
    inter = iw * ih
    ua = jnp.clip(aw * ah + area_b - inter, 1e-8, None)
    iou = inter / ua

    iou_max = jnp.max(iou, axis=0, keepdims=True)
    iota_s = lax.broadcasted_iota(jnp.int32, iou.shape, 0)
    amax = jnp.min(jnp.where(iou == iou_max, iota_s, 32),
                   axis=0, keepdims=True)
    onehot = (iota_s == amax).astype(jnp.float32)

    pos = iou_max >= 0.5
    posf = pos.astype(jnp.float32)
    ignoref = ((iou_max >= 0.4) & (iou_max < 0.5)).astype(jnp.float32)
    validf = 1.0 - ignoref

    bw = x2 - x1
    bh = y2 - y1
    bcx = x1 + 0.5 * bw
    bcy = y1 + 0.5 * bh
    lgw = jnp.log(jnp.clip(bw, 1.0, None))
    lgh = jnp.log(jnp.clip(bh, 1.0, None))
    valmat = jnp.concatenate([bcx, bcy, lgw, lgh], axis=1)
    assigned = lax.dot_general(valmat, onehot, _CONTRACT0,
                               precision=_HI,
                               preferred_element_type=jnp.float32)

    tdx = (assigned[0:1, :] - acx) / aw * 10.0
    tdy = (assigned[1:2, :] - acy) / ah * 10.0
    tdw = (assigned[2:3, :] - jnp.log(aw)) * 5.0
    tdh = (assigned[3:4, :] - jnp.log(ah)) * 5.0
    reg = reg_ref[0, 0]
    d0 = reg[0:1, :] - tdx
    d1 = reg[1:2, :] - tdy
    d2 = reg[2:3, :] - tdw
    d3 = reg[3:4, :] - tdh
    sq = d0 * d0 + d1 * d1 + d2 * d2 + d3 * d3
    reg_blk = jnp.sum(sq * posf)
    npos_blk = jnp.sum(posf)

    iota_c = lax.broadcasted_iota(jnp.int32, (32, 80), 1)
    clsmat = (bcls == iota_c.astype(jnp.float32)).astype(jnp.float32)
    onehot_pos = onehot * posf
    lhs = jnp.concatenate([onehot_pos, ignoref], axis=0)
    rhs = jnp.concatenate([0.5 * clsmat,
                           jnp.full((1, 80), 0.75, jnp.float32)], axis=0)
    u = lax.dot_general(lhs.astype(jnp.bfloat16), rhs.astype(jnp.bfloat16),
                        _CONTRACT0,
                        preferred_element_type=jnp.float32)
    af_eff = 0.75 - u
    is_one = af_eff == 0.25

    p = jnp.clip(cls_ref[0], 1e-4, 1.0 - 1e-4)
    q = jnp.where(is_one, p, 1.0 - p)
    fw = 1.0 - q
    contrib = af_eff * (fw * fw) * jnp.log(q)
    cls_blk = -jnp.sum(contrib)

    @pl.when(b == 0)
    def _():
        cls_out[0, 0, 0] = cls_blk
        reg_out[0, 0, 0] = reg_blk
        npos_out[0, 0, 0] = npos_blk

    @pl.when(b != 0)
    def _():
        cls_out[0, 0, 0] += cls_blk
        reg_out[0, 0, 0] += reg_blk
        npos_out[0, 0, 0] += npos_blk


@jax.jit
def kernel(classifications, regressions, anchors, annotations):
    B, A, C = classifications.shape
    nb = A // BLK
    anc_t = anchors[0].reshape(nb, BLK, 4).transpose(0, 2, 1)
    reg_t = regressions.reshape(B, nb, BLK, 4).transpose(0, 1, 3, 2)

    out_shapes = [
        jax.ShapeDtypeStruct((B, 1, 1), jnp.float32),
        jax.ShapeDtypeStruct((B, 1, 1), jnp.float32),
        jax.ShapeDtypeStruct((B, 1, 1), jnp.float32),
    ]
    cls_sum, reg_sum, npos = pl.pallas_call(
        _focal_body,
        grid=(B, nb),
        in_specs=[
            pl.BlockSpec((1, BLK, C), lambda j, b: (j, b, 0)),
            pl.BlockSpec((1, 1, 4, BLK), lambda j, b: (j, b, 0, 0)),
            pl.BlockSpec((1, 4, BLK), lambda j, b: (b, 0, 0)),
            pl.BlockSpec((1, 32, 5), lambda j, b: (j, 0, 0)),
        ],
        out_specs=[
            pl.BlockSpec((1, 1, 1), lambda j, b: (j, 0, 0),
                         memory_space=pltpu.SMEM),
            pl.BlockSpec((1, 1, 1), lambda j, b: (j, 0, 0),
                         memory_space=pltpu.SMEM),
            pl.BlockSpec((1, 1, 1), lambda j, b: (j, 0, 0),
                         memory_space=pltpu.SMEM),
        ],
        out_shape=out_shapes,
        compiler_params=pltpu.CompilerParams(
            dimension_semantics=("parallel", "arbitrary"),
        ),
    )(classifications, reg_t, anc_t, annotations)

    npos = npos[:, 0, 0]
    cls_loss = cls_sum[:, 0, 0] / jnp.maximum(npos, 1.0)
    reg_loss = reg_sum[:, 0, 0] / jnp.maximum(npos * 4.0, 1.0) * 2.0
    reg_loss = jnp.where(npos > 0.0, reg_loss, 0.0)
    return (jnp.mean(cls_loss, keepdims=True),
            jnp.mean(reg_loss, keepdims=True))


# speedup vs baseline: 1.1047x; 1.1047x over previous
import functools

import jax
import jax.numpy as jnp
from jax import lax
from jax.experimental import pallas as pl
from jax.experimental.pallas import tpu as pltpu

ALPHA = 0.25
BLK = 20000

_CONTRACT0 = (((0,), (0,)), ((), ()))
_HI = lax.Precision.HIGHEST


def _focal_body(cls_ref, reg_ref, anc_ref, ann_ref,
                cls_out, reg_out, npos_out):
    b = pl.program_id(1)

    ann = ann_ref[0]
    x1 = ann[:, 0:1]
    y1 = ann[:, 1:2]
    x2 = ann[:, 2:3]
    y2 = ann[:, 3:4]
    bcls = ann[:, 4:5]
    area_b = (x2 - x1) * (y2 - y1)

    anc = anc_ref[0]
    ax1 = anc[0:1, :]
    ay1 = anc[1:2, :]
    ax2 = anc[2:3, :]
    ay2 = anc[3:4, :]
    aw = ax2 - ax1
    ah = ay2 - ay1
    acx = ax1 + 0.5 * aw
    acy = ay1 + 0.5 * ah

    iw = jnp.clip(jnp.minimum(ax2, x2) - jnp.maximum(ax1, x1), 0.0, None)
    ih = jnp.clip(jnp.minimum(ay2, y2) - jnp.maximum(ay1, y1), 0.0, None)
    inter = iw * ih
    ua = jnp.clip(aw * ah + area_b - inter, 1e-8, None)
    iou = inter / ua

    iou_max = jnp.max(iou, axis=0, keepdims=True)
    iota_s = lax.broadcasted_iota(jnp.int32, iou.shape, 0)
    amax = jnp.min(jnp.where(iou == iou_max, iota_s, 32),
                   axis=0, keepdims=True)
    onehot = (iota_s == amax).astype(jnp.float32)

    pos = iou_max >= 0.5
    posf = pos.astype(jnp.float32)
    ignoref = ((iou_max >= 0.4) & (iou_max < 0.5)).astype(jnp.float32)
    validf = 1.0 - ignoref

    bw = x2 - x1
    bh = y2 - y1
    bcx = x1 + 0.5 * bw
    bcy = y1 + 0.5 * bh
    lgw = jnp.log(jnp.clip(bw, 1.0, None))
    lgh = jnp.log(jnp.clip(bh, 1.0, None))
    valmat = jnp.concatenate([bcx, bcy, lgw, lgh], axis=1)
    assigned = lax.dot_general(valmat, onehot, _CONTRACT0,
                               preferred_element_type=jnp.float32)

    tdx = (assigned[0:1, :] - acx) / aw * 10.0
    tdy = (assigned[1:2, :] - acy) / ah * 10.0
    tdw = (assigned[2:3, :] - jnp.log(aw)) * 5.0
    tdh = (assigned[3:4, :] - jnp.log(ah)) * 5.0
    reg = reg_ref[0, 0]
    d0 = reg[0:1, :] - tdx
    d1 = reg[1:2, :] - tdy
    d2 = reg[2:3, :] - tdw
    d3 = reg[3:4, :] - tdh
    sq = d0 * d0 + d1 * d1 + d2 * d2 + d3 * d3
    reg_blk = jnp.sum(sq * posf)
    npos_blk = jnp.sum(posf)

    iota_c = lax.broadcasted_iota(jnp.int32, (32, 80), 1)
    clsmat = (bcls == iota_c.astype(jnp.float32)).astype(jnp.float32)
    onehot_pos = onehot * posf
    ones_row = jnp.ones((1, BLK), jnp.float32)
    lhs = jnp.concatenate([onehot_pos, ignoref, ones_row], axis=0)
    rhs = jnp.concatenate([-0.5 * clsmat,
                           jnp.full((1, 80), -0.75, jnp.float32),
                           jnp.full((1, 80), 0.75, jnp.float32)], axis=0)
    af_eff = lax.dot_general(lhs.astype(jnp.bfloat16),
                             rhs.astype(jnp.bfloat16), _CONTRACT0,
                             preferred_element_type=jnp.float32)
    is_one = af_eff == 0.25

    p = jnp.clip(cls_ref[0], 1e-4, 1.0 - 1e-4)
    q = jnp.where(is_one, p, 1.0 - p)
    fw = 1.0 - q
    contrib = af_eff * (fw * fw) * jnp.log(q)
    cls_blk = -jnp.sum(contrib)

    @pl.when(b == 0)
    def _():
        cls_out[0, 0, 0] = cls_blk
        reg_out[0, 0, 0] = reg_blk
        npos_out[0, 0, 0] = npos_blk

    @pl.when(b != 0)
    def _():
        cls_out[0, 0, 0] += cls_blk
        reg_out[0, 0, 0] += reg_blk
        npos_out[0, 0, 0] += npos_blk


@jax.jit
def kernel(classifications, regressions, anchors, annotations):
    B, A, C = classifications.shape
    nb = A // BLK
    anc_t = anchors[0].reshape(nb, BLK, 4).transpose(0, 2, 1)
    reg_t = regressions.reshape(B, nb, BLK, 4).transpose(0, 1, 3, 2)

    out_shapes = [
        jax.ShapeDtypeStruct((B, 1, 1), jnp.float32),
        jax.ShapeDtypeStruct((B, 1, 1), jnp.float32),
        jax.ShapeDtypeStruct((B, 1, 1), jnp.float32),
    ]
    cls_sum, reg_sum, npos = pl.pallas_call(
        _focal_body,
        grid=(B, nb),
        in_specs=[
            pl.BlockSpec((1, BLK, C), lambda j, b: (j, b, 0)),
            pl.BlockSpec((1, 1, 4, BLK), lambda j, b: (j, b, 0, 0)),
            pl.BlockSpec((1, 4, BLK), lambda j, b: (b, 0, 0)),
            pl.BlockSpec((1, 32, 5), lambda j, b: (j, 0, 0)),
        ],
        out_specs=[
            pl.BlockSpec((1, 1, 1), lambda j, b: (j, 0, 0),
                         memory_space=pltpu.SMEM),
            pl.BlockSpec((1, 1, 1), lambda j, b: (j, 0, 0),
                         memory_space=pltpu.SMEM),
            pl.BlockSpec((1, 1, 1), lambda j, b: (j, 0, 0),
                         memory_space=pltpu.SMEM),
        ],
        out_shape=out_shapes,
        compiler_params=pltpu.CompilerParams(
            dimension_semantics=("parallel", "arbitrary"),
        ),
    )(classifications, reg_t, anc_t, annotations)

    npos = npos[:, 0, 0]
    cls_loss = cls_sum[:, 0, 0] / jnp.maximum(npos, 1.0)
    reg_loss = reg_sum[:, 0, 0] / jnp.maximum(npos * 4.0, 1.0) * 2.0
    reg_loss = jnp.where(npos > 0.0, reg_loss, 0.0)
    return (jnp.mean(cls_loss, keepdims=True),
            jnp.mean(reg_loss, keepdims=True))
